# baseline (speedup 1.0000x reference)
import functools

import jax
import jax.numpy as jnp
from jax import lax
from jax.experimental import pallas as pl
from jax.experimental.pallas import tpu as pltpu
from jax.experimental.pallas import tpu_sc as plsc

D_MODEL = 64
SCALE = 8.0
NC, NS = 2, 16
NW = NC * NS
IDXW = 128
CHUNK = 512
JR = CHUNK // IDXW


@functools.lru_cache(maxsize=None)
def _make_sc_embed(B, V):
    b_per_w = B // NW
    G = b_per_w // CHUNK
    idx_rows_w = b_per_w // IDXW
    assert B % (NW * CHUNK) == 0 and G % 2 == 0 and G >= 2

    mesh = plsc.VectorSubcoreMesh(core_axis_name="c", subcore_axis_name="s",
                                  num_cores=NC, num_subcores=NS)

    @functools.partial(
        pl.kernel,
        out_type=jax.ShapeDtypeStruct((B, D_MODEL), jnp.float32),
        mesh=mesh,
        scratch_types=[
            pltpu.VMEM((idx_rows_w, IDXW), jnp.int32),
            pltpu.VMEM((CHUNK, D_MODEL), jnp.float32),
            pltpu.VMEM((CHUNK, D_MODEL), jnp.float32),
            pltpu.SemaphoreType.DMA,
            pltpu.SemaphoreType.DMA,
            pltpu.SemaphoreType.DMA,
            pltpu.SemaphoreType.DMA,
        ],
        compiler_params=pltpu.CompilerParams(use_tc_tiling_on_sc=False),
    )
    def embed(x_hbm, tab_hbm, out_hbm, idx_v, rows0, rows1,
              g0sem, g1sem, o0sem, o1sem):
        rows = (rows0, rows1)
        gsem = (g0sem, g1sem)
        osem = (o0sem, o1sem)
        wid = lax.axis_index("s") * NC + lax.axis_index("c")
        ibase = wid * idx_rows_w
        obase = wid * b_per_w

        pltpu.sync_copy(x_hbm.at[pl.ds(ibase, idx_rows_w)], idx_v)

        def fire_gather(g, b):
            for j in range(JR):
                pltpu.async_copy(
                    tab_hbm.at[idx_v.at[g * JR + j]],
                    rows[b].at[pl.ds(j * IDXW, IDXW)],
                    gsem[b])

        def wait_gather(g, b):
            for j in range(JR):
                pltpu.make_async_copy(
                    tab_hbm.at[idx_v.at[g * JR + j]],
                    rows[b].at[pl.ds(j * IDXW, IDXW)],
                    gsem[b]).wait()

        def fire_out(g, b):
            pltpu.async_copy(
                rows[b], out_hbm.at[pl.ds(obase + g * CHUNK, CHUNK)], osem[b])

        def wait_out(g, b):
            pltpu.make_async_copy(
                rows[b], out_hbm.at[pl.ds(obase + g * CHUNK, CHUNK)],
                osem[b]).wait()

        def scale(b):
            r = rows[b]

            @pl.loop(0, CHUNK, unroll=4)
            def _(i):
                for c in range(D_MODEL // 16):
                    sl = pl.ds(c * 16, 16)
                    r[i, sl] = r[i, sl] * SCALE

        fire_gather(0, 0)

        @pl.loop(0, G, step=2)
        def _(g0):
            for b in range(2):
                g = g0 + b
                nb = 1 - b

                @pl.when(g + 1 < G)
                def _():
                    @pl.when(g >= 1)
                    def _():
                        wait_out(g - 1, nb)

                    fire_gather(g + 1, nb)

                wait_gather(g, b)
                scale(b)
                fire_out(g, b)

        wait_out(G - 2, 0)
        wait_out(G - 1, 1)

    return embed


@jax.jit
def kernel(x, table):
    s0, s1 = x.shape
    B = s0 * s1
    x2d = x.reshape(B // IDXW, IDXW).astype(jnp.int32)
    out = _make_sc_embed(B, table.shape[0])(x2d, table)
    return out.reshape(s0, s1, D_MODEL)
